# Initial kernel scaffold; baseline (speedup 1.0000x reference)
import jax
import jax.numpy as jnp
from jax.experimental import pallas as pl


def kernel(g, X, emb, k, W_o, b_o, a_o, W_c, b_c, a_c):
    raise NotImplementedError("write your pallas kernel here")


# speedup vs baseline: 5.8670x; 5.8670x over previous
import functools

import jax
import jax.numpy as jnp
from jax import lax
from jax.experimental import pallas as pl
from jax.experimental.pallas import tpu as pltpu
from jax.experimental.pallas import tpu_sc as plsc

N = 10000
E = 320000
D = 128
NC = 2
NS = 16
NW = NC * NS
EPW = E // NW
C = 80
NCHUNK = EPW // C
NPAD = 10240
RPT = NPAD // NS


def _sc_hop(want_deg: bool, h_rows: int):
    mesh = plsc.VectorSubcoreMesh(core_axis_name="c", subcore_axis_name="s")

    out_type = [jax.ShapeDtypeStruct((NC, NPAD, D), jnp.float32)]
    scratch = [
        pltpu.VMEM_SHARED((NPAD, D), jnp.float32),
        pltpu.VMEM((C,), jnp.int32),
        pltpu.VMEM((C,), jnp.int32),
        pltpu.VMEM((C, D), jnp.float32),
        pltpu.SemaphoreType.DMA,
    ]
    if want_deg:
        out_type.append(jax.ShapeDtypeStruct((NC, NPAD), jnp.float32))
        scratch += [
            pltpu.VMEM_SHARED((NPAD,), jnp.float32),
            pltpu.VMEM((C,), jnp.float32),
            pltpu.VMEM((RPT,), jnp.float32),
        ]

    def body(h_hbm, src_hbm, dst_hbm, *rest):
        if want_deg:
            (s_out, deg_out, accum_sh, src_v, dst_v, rows_v, sem,
             deg_sh, ones_v, zdeg_v) = rest
        else:
            (s_out, accum_sh, src_v, dst_v, rows_v, sem) = rest
        c = lax.axis_index("c")
        s = lax.axis_index("s")
        wid = c * NS + s
        r0 = s * RPT

        z16 = jnp.zeros((16,), jnp.float32)

        def zrow(i, _):
            def zcol(j, _):
                rows_v[i, pl.ds(j * 16, 16)] = z16
                return 0
            return lax.fori_loop(0, D // 16, zcol, 0)
        lax.fori_loop(0, C, zrow, 0)
        for j in range(RPT // C):
            pltpu.sync_copy(rows_v, accum_sh.at[pl.ds(r0 + j * C, C)])

        if want_deg:
            o16 = jnp.ones((16,), jnp.float32)
            def fill1(i, _):
                ones_v[pl.ds(i * 16, 16)] = o16
                return 0
            lax.fori_loop(0, C // 16, fill1, 0)
            def fill0(i, _):
                zdeg_v[pl.ds(i * 16, 16)] = z16
                return 0
            lax.fori_loop(0, RPT // 16, fill0, 0)
            pltpu.sync_copy(zdeg_v, deg_sh.at[pl.ds(r0, RPT)])

        plsc.subcore_barrier()

        def edge(i, _):
            pltpu.sync_copy(src_hbm.at[wid, i], src_v)
            pltpu.sync_copy(dst_hbm.at[wid, i], dst_v)
            pltpu.async_copy(h_hbm.at[src_v], rows_v, sem).wait()
            pltpu.sync_copy(rows_v, accum_sh.at[dst_v], add=True)
            if want_deg:
                pltpu.sync_copy(ones_v, deg_sh.at[dst_v], add=True)
            return 0
        lax.fori_loop(0, NCHUNK, edge, 0)

        plsc.subcore_barrier()

        pltpu.sync_copy(accum_sh.at[pl.ds(r0, RPT)], s_out.at[c, pl.ds(r0, RPT)])
        if want_deg:
            pltpu.sync_copy(deg_sh.at[pl.ds(r0, RPT)], deg_out.at[c, pl.ds(r0, RPT)])

    return pl.kernel(body, out_type=tuple(out_type), mesh=mesh,
                     scratch_types=tuple(scratch))


def _merge_body(s_ref, d_ref, h_ref):
    deg = jnp.clip(d_ref[0] + d_ref[1], 1.0, None)
    h_ref[...] = (s_ref[0] + s_ref[1]) / deg


def _merge(s_part, deg_part):
    BM = 512
    grid = (NPAD // BM,)
    return pl.pallas_call(
        _merge_body,
        grid=grid,
        in_specs=[
            pl.BlockSpec((NC, BM, D), lambda i: (0, i, 0)),
            pl.BlockSpec((NC, BM, 1), lambda i: (0, i, 0)),
        ],
        out_specs=pl.BlockSpec((BM, D), lambda i: (i, 0)),
        out_shape=jax.ShapeDtypeStruct((NPAD, D), jnp.float32),
    )(s_part, deg_part)


def _prelu(x, a):
    return jnp.maximum(x, 0.0) + a * jnp.minimum(x, 0.0)


def _final_body(s_ref, d_ref, x_ref, emb_ref, wc_ref, bc_ref, ac_ref,
                wo_ref, bo_ref, ao_ref, o_ref):
    deg = jnp.clip(d_ref[0] + d_ref[1], 1.0, None)
    agg = (s_ref[0] + s_ref[1]) / deg
    x = x_ref[...]
    for l in range(2):
        h = (jnp.dot(x, wc_ref[l, :D, :], preferred_element_type=jnp.float32)
             + jnp.dot(agg, wc_ref[l, D:, :], preferred_element_type=jnp.float32)
             + bc_ref[l][None, :])
        agg = _prelu(h, ac_ref[l][None, :])
    res = jnp.dot(emb_ref[...], wo_ref[...], preferred_element_type=jnp.float32)
    res = _prelu(res + bo_ref[...], ao_ref[...])
    o_ref[...] = agg + res


def _final(s_part, deg_part, X, emb, W_c, b_c, a_c, W_o, b_o, a_o):
    BN = 400
    grid = (N // BN,)
    full = lambda *shape: pl.BlockSpec(shape, lambda i: (0,) * len(shape))
    return pl.pallas_call(
        _final_body,
        grid=grid,
        in_specs=[
            pl.BlockSpec((NC, BN, D), lambda i: (0, i, 0)),
            pl.BlockSpec((NC, BN, 1), lambda i: (0, i, 0)),
            pl.BlockSpec((BN, D), lambda i: (i, 0)),
            pl.BlockSpec((BN, D), lambda i: (i, 0)),
            full(2, 2 * D, D),
            full(2, D),
            full(2, D),
            full(D, D),
            full(1, D),
            full(1, D),
        ],
        out_specs=pl.BlockSpec((BN, D), lambda i: (i, 0)),
        out_shape=jax.ShapeDtypeStruct((N, D), jnp.float32),
    )(s_part, deg_part, X, emb, W_c, b_c, a_c, W_o, b_o, a_o)


def kernel(g, X, emb, k, W_o, b_o, a_o, W_c, b_c, a_c):
    src3 = g[0].astype(jnp.int32).reshape(NW, NCHUNK, C)
    dst3 = g[1].astype(jnp.int32).reshape(NW, NCHUNK, C)

    hop_deg = _sc_hop(True, N)
    hop_nodeg = _sc_hop(False, NPAD)

    s1, deg = hop_deg(X, src3, dst3)
    deg = deg.reshape(NC, NPAD, 1)
    h1 = _merge(s1, deg)
    (s2,) = hop_nodeg(h1, src3, dst3)

    ac_b = jnp.broadcast_to(a_c[:, None], (2, D))
    bo_b = jnp.reshape(b_o, (1, D))
    ao_b = jnp.full((1, D), a_o, dtype=jnp.float32)
    return _final(s2, deg, X, emb, W_c, b_c, ac_b, W_o, bo_b, ao_b)
